# baseline (speedup 1.0000x reference)
import jax
import jax.numpy as jnp
from jax import lax
from jax.experimental import pallas as pl
from jax.experimental.pallas import tpu as pltpu
from jax.experimental.pallas import tpu_sc as plsc

NODE_NUM = 100000
EMBED_DIM = 128
BATCH = 16384
HIST = 50

NC = 2
NS = 16
NW = NC * NS

B_W = BATCH // NW
KB = 2
CHUNK = KB * HIST
NCHUNK = B_W // KB
IDX_PAD = 128

NBUF = 4
PREF = 2


def _body(idx_hbm, w_hbm, out_hbm, idx_v, buf_v, gsems, wsems):
    wid = lax.axis_index("s") * NC + lax.axis_index("c")
    b0 = wid * B_W
    pltpu.sync_copy(idx_hbm.at[wid], idx_v)

    def gather(g, bg):
        pltpu.make_async_copy(
            w_hbm.at[idx_v.at[g, pl.ds(0, CHUNK)]], buf_v.at[bg], gsems.at[bg]
        ).start()

    def write(j, b, start):
        for i in range(KB):
            cp = pltpu.make_async_copy(
                buf_v.at[b, pl.ds(i * HIST, HIST)],
                out_hbm.at[b0 + j * KB + i],
                wsems.at[b],
            )
            cp.start() if start else cp.wait()

    for b in range(PREF):
        gather(b, b)

    def step(j, carry):
        b = lax.rem(j, NBUF)
        pltpu.make_async_copy(
            w_hbm.at[idx_v.at[j, pl.ds(0, CHUNK)]], buf_v.at[b], gsems.at[b]
        ).wait()
        write(j, b, start=True)
        g = j + PREF

        @pl.when(g < NCHUNK)
        def _():
            bg = lax.rem(g, NBUF)

            @pl.when(g >= NBUF)
            def _():
                write(g - NBUF, bg, start=False)

            gather(g, bg)

        return carry

    lax.fori_loop(0, NCHUNK, step, 0)

    for t in range(NBUF):
        j = NCHUNK - NBUF + t
        write(j, j % NBUF, start=False)


@jax.jit
def _run(targ, W):
    idx = targ.reshape(NW, NCHUNK, CHUNK)
    idx = jnp.pad(idx, ((0, 0), (0, 0), (0, IDX_PAD - CHUNK)))
    mesh = plsc.VectorSubcoreMesh(core_axis_name="c", subcore_axis_name="s")
    k = pl.kernel(
        _body,
        out_type=jax.ShapeDtypeStruct((BATCH, HIST, EMBED_DIM), jnp.float32),
        mesh=mesh,
        compiler_params=pltpu.CompilerParams(use_tc_tiling_on_sc=True),
        scratch_types=[
            pltpu.VMEM((NCHUNK, IDX_PAD), jnp.int32),
            pltpu.VMEM((NBUF, CHUNK, EMBED_DIM), jnp.float32),
            pltpu.SemaphoreType.DMA((NBUF,)),
            pltpu.SemaphoreType.DMA((NBUF,)),
        ],
    )
    return k(idx, W)


def kernel(targ, W):
    return _run(targ.astype(jnp.int32), W)


# speedup vs baseline: 1.9025x; 1.9025x over previous
import jax
import jax.numpy as jnp
from jax import lax
from jax.experimental import pallas as pl
from jax.experimental.pallas import tpu as pltpu
from jax.experimental.pallas import tpu_sc as plsc

NODE_NUM = 100000
EMBED_DIM = 128
BATCH = 16384
HIST = 50

NC = 2
NS = 16
NW = NC * NS

TOTAL = BATCH * HIST
PER_W = TOTAL // NW
CHUNK = 128
NCHUNK = PER_W // CHUNK

NBUF = 6
PREF = 3


def _body(idx_hbm, w_hbm, out_hbm, idx_v, buf_v, gsems, wsems):
    wid = lax.axis_index("s") * NC + lax.axis_index("c")
    base = wid * PER_W
    out_flat = out_hbm.reshape(TOTAL, EMBED_DIM)
    pltpu.sync_copy(idx_hbm.at[wid], idx_v)

    def gather(g, bg):
        pltpu.make_async_copy(
            w_hbm.at[idx_v.at[g]], buf_v.at[bg], gsems.at[bg]
        ).start()

    for b in range(PREF):
        gather(b, b)

    def step(j, carry):
        b = lax.rem(j, NBUF)
        pltpu.make_async_copy(
            w_hbm.at[idx_v.at[j]], buf_v.at[b], gsems.at[b]
        ).wait()
        pltpu.make_async_copy(
            buf_v.at[b], out_flat.at[pl.ds(base + j * CHUNK, CHUNK)], wsems.at[b]
        ).start()
        g = j + PREF

        @pl.when(g < NCHUNK)
        def _():
            bg = lax.rem(g, NBUF)

            @pl.when(g >= NBUF)
            def _():
                pltpu.make_async_copy(
                    buf_v.at[bg],
                    out_flat.at[pl.ds(base + (g - NBUF) * CHUNK, CHUNK)],
                    wsems.at[bg],
                ).wait()

            gather(g, bg)

        return carry

    lax.fori_loop(0, NCHUNK, step, 0)

    for t in range(NBUF):
        j = NCHUNK - NBUF + t
        b = j % NBUF
        pltpu.make_async_copy(
            buf_v.at[b], out_flat.at[pl.ds(base + j * CHUNK, CHUNK)], wsems.at[b]
        ).wait()


@jax.jit
def _run(targ, W):
    idx = targ.T.reshape(NW, NCHUNK, CHUNK)
    mesh = plsc.VectorSubcoreMesh(core_axis_name="c", subcore_axis_name="s")
    k = pl.kernel(
        _body,
        out_type=jax.ShapeDtypeStruct((HIST, BATCH, EMBED_DIM), jnp.float32),
        mesh=mesh,
        compiler_params=pltpu.CompilerParams(use_tc_tiling_on_sc=True),
        scratch_types=[
            pltpu.VMEM((NCHUNK, CHUNK), jnp.int32),
            pltpu.VMEM((NBUF, CHUNK, EMBED_DIM), jnp.float32),
            pltpu.SemaphoreType.DMA((NBUF,)),
            pltpu.SemaphoreType.DMA((NBUF,)),
        ],
    )
    out_t = k(idx, W)
    return jnp.transpose(out_t, (1, 0, 2))


def kernel(targ, W):
    return _run(targ.astype(jnp.int32), W)


# speedup vs baseline: 1.9245x; 1.0116x over previous
import jax
import jax.numpy as jnp
from jax import lax
from jax.experimental import pallas as pl
from jax.experimental.pallas import tpu as pltpu
from jax.experimental.pallas import tpu_sc as plsc

NODE_NUM = 100000
EMBED_DIM = 128
BATCH = 16384
HIST = 50

NC = 2
NS = 16
NW = NC * NS

TOTAL = BATCH * HIST
PER_W = TOTAL // NW
CHUNK = 128
NCHUNK = PER_W // CHUNK
CPH = BATCH // CHUNK

NBUF = 6
PREF = 3
PREF2 = 5
NIDX = 8


def _body(tt_hbm, w_hbm, out_hbm, idx_v, buf_v, isems, gsems, wsems):
    wid = lax.axis_index("s") * NC + lax.axis_index("c")
    base = wid * PER_W
    out_flat = out_hbm.reshape(TOTAL, EMBED_DIM)

    def stage(c):
        gc = wid * NCHUNK + c
        h = lax.div(gc, CPH)
        b0 = lax.rem(gc, CPH) * CHUNK
        s = lax.rem(c, NIDX)
        pltpu.make_async_copy(
            tt_hbm.at[h, pl.ds(b0, CHUNK)], idx_v.at[s], isems.at[s]
        ).start()

    def gather(g, bg):
        s = lax.rem(g, NIDX)
        pltpu.make_async_copy(
            tt_hbm.at[0, pl.ds(0, CHUNK)], idx_v.at[s], isems.at[s]
        ).wait()
        pltpu.make_async_copy(
            w_hbm.at[idx_v.at[s]], buf_v.at[bg], gsems.at[bg]
        ).start()

    for c in range(PREF2):
        stage(c)
    for g in range(PREF):
        gather(g, g)

    def step(j, carry):
        b = lax.rem(j, NBUF)
        pltpu.make_async_copy(
            w_hbm.at[idx_v.at[0]], buf_v.at[b], gsems.at[b]
        ).wait()
        pltpu.make_async_copy(
            buf_v.at[b], out_flat.at[pl.ds(base + j * CHUNK, CHUNK)], wsems.at[b]
        ).start()
        g = j + PREF

        @pl.when(g < NCHUNK)
        def _():
            bg = lax.rem(g, NBUF)

            @pl.when(g >= NBUF)
            def _():
                pltpu.make_async_copy(
                    buf_v.at[bg],
                    out_flat.at[pl.ds(base + (g - NBUF) * CHUNK, CHUNK)],
                    wsems.at[bg],
                ).wait()

            gather(g, bg)

        c = j + PREF2

        @pl.when(c < NCHUNK)
        def _():
            stage(c)

        return carry

    lax.fori_loop(0, NCHUNK, step, 0)

    for t in range(NBUF):
        j = NCHUNK - NBUF + t
        b = j % NBUF
        pltpu.make_async_copy(
            buf_v.at[b], out_flat.at[pl.ds(base + j * CHUNK, CHUNK)], wsems.at[b]
        ).wait()


@jax.jit
def _run(targ, W):
    tt = targ.T
    mesh = plsc.VectorSubcoreMesh(core_axis_name="c", subcore_axis_name="s")
    k = pl.kernel(
        _body,
        out_type=jax.ShapeDtypeStruct((HIST, BATCH, EMBED_DIM), jnp.float32),
        mesh=mesh,
        compiler_params=pltpu.CompilerParams(use_tc_tiling_on_sc=True),
        scratch_types=[
            pltpu.VMEM((NIDX, CHUNK), jnp.int32),
            pltpu.VMEM((NBUF, CHUNK, EMBED_DIM), jnp.float32),
            pltpu.SemaphoreType.DMA((NIDX,)),
            pltpu.SemaphoreType.DMA((NBUF,)),
            pltpu.SemaphoreType.DMA((NBUF,)),
        ],
    )
    out_t = k(tt, W)
    return jnp.transpose(out_t, (1, 0, 2))


def kernel(targ, W):
    return _run(targ.astype(jnp.int32), W)


# speedup vs baseline: 1.9287x; 1.0022x over previous
import jax
import jax.numpy as jnp
from jax import lax
from jax.experimental import pallas as pl
from jax.experimental.pallas import tpu as pltpu
from jax.experimental.pallas import tpu_sc as plsc

NODE_NUM = 100000
EMBED_DIM = 128
BATCH = 16384
HIST = 50

NC = 2
NS = 16
NW = NC * NS

TOTAL = BATCH * HIST
PER_W = TOTAL // NW
CHUNK = 128
NCHUNK = PER_W // CHUNK
CPH = BATCH // CHUNK

NBUF = 7
PREF = 4
PREF2 = 6
NIDX = 8


def _body(tt_hbm, w_hbm, out_hbm, idx_v, buf_v, isems, gsems, wsems):
    wid = lax.axis_index("s") * NC + lax.axis_index("c")
    base = wid * PER_W
    out_flat = out_hbm.reshape(TOTAL, EMBED_DIM)

    def stage(c):
        gc = wid * NCHUNK + c
        h = lax.div(gc, CPH)
        b0 = lax.rem(gc, CPH) * CHUNK
        s = lax.rem(c, NIDX)
        pltpu.make_async_copy(
            tt_hbm.at[h, pl.ds(b0, CHUNK)], idx_v.at[s], isems.at[s]
        ).start()

    def gather(g, bg):
        s = lax.rem(g, NIDX)
        pltpu.make_async_copy(
            tt_hbm.at[0, pl.ds(0, CHUNK)], idx_v.at[s], isems.at[s]
        ).wait()
        pltpu.make_async_copy(
            w_hbm.at[idx_v.at[s]], buf_v.at[bg], gsems.at[bg]
        ).start()

    for c in range(PREF2):
        stage(c)
    for g in range(PREF):
        gather(g, g)

    def step(j, carry):
        b = lax.rem(j, NBUF)
        pltpu.make_async_copy(
            w_hbm.at[idx_v.at[0]], buf_v.at[b], gsems.at[b]
        ).wait()
        pltpu.make_async_copy(
            buf_v.at[b], out_flat.at[pl.ds(base + j * CHUNK, CHUNK)], wsems.at[b]
        ).start()
        g = j + PREF

        @pl.when(g < NCHUNK)
        def _():
            bg = lax.rem(g, NBUF)

            @pl.when(g >= NBUF)
            def _():
                pltpu.make_async_copy(
                    buf_v.at[bg],
                    out_flat.at[pl.ds(base + (g - NBUF) * CHUNK, CHUNK)],
                    wsems.at[bg],
                ).wait()

            gather(g, bg)

        c = j + PREF2

        @pl.when(c < NCHUNK)
        def _():
            stage(c)

        return carry

    lax.fori_loop(0, NCHUNK, step, 0)

    for t in range(NBUF):
        j = NCHUNK - NBUF + t
        b = j % NBUF
        pltpu.make_async_copy(
            buf_v.at[b], out_flat.at[pl.ds(base + j * CHUNK, CHUNK)], wsems.at[b]
        ).wait()


@jax.jit
def _run(targ, W):
    tt = targ.T
    mesh = plsc.VectorSubcoreMesh(core_axis_name="c", subcore_axis_name="s")
    k = pl.kernel(
        _body,
        out_type=jax.ShapeDtypeStruct((HIST, BATCH, EMBED_DIM), jnp.float32),
        mesh=mesh,
        compiler_params=pltpu.CompilerParams(use_tc_tiling_on_sc=True),
        scratch_types=[
            pltpu.VMEM((NIDX, CHUNK), jnp.int32),
            pltpu.VMEM((NBUF, CHUNK, EMBED_DIM), jnp.float32),
            pltpu.SemaphoreType.DMA((NIDX,)),
            pltpu.SemaphoreType.DMA((NBUF,)),
            pltpu.SemaphoreType.DMA((NBUF,)),
        ],
    )
    out_t = k(tt, W)
    return jnp.transpose(out_t, (1, 0, 2))


def kernel(targ, W):
    return _run(targ.astype(jnp.int32), W)
